# Initial kernel scaffold; baseline (speedup 1.0000x reference)
import jax
import jax.numpy as jnp
from jax.experimental import pallas as pl


def kernel(x, edge_index, W1, b1, W2, b2, Wl, bl):
    raise NotImplementedError("write your pallas kernel here")


# speedup vs baseline: 11.9119x; 11.9119x over previous
import functools

import jax
import jax.numpy as jnp
from jax import lax
from jax.experimental import pallas as pl
from jax.experimental.pallas import tpu as pltpu
from jax.experimental.pallas import tpu_sc as plsc

NC = 2
NS = 16
NW = NC * NS
CHUNK = 128


def _round_up(a, m):
    return (a + m - 1) // m * m


def _make_deg(np_, rp, nchunk):
    mesh = plsc.VectorSubcoreMesh(core_axis_name="c", subcore_axis_name="s")

    @functools.partial(
        pl.kernel,
        out_type=jax.ShapeDtypeStruct((NC, np_, 16), jnp.float32),
        mesh=mesh,
        compiler_params=pltpu.CompilerParams(use_tc_tiling_on_sc=False),
        scratch_types=[
            pltpu.VMEM((nchunk, CHUNK), jnp.int32),
            pltpu.VMEM((CHUNK, 16), jnp.float32),
            pltpu.VMEM_SHARED((np_, 16), jnp.float32),
        ],
    )
    def deg_kernel(dst_hbm, e0_hbm, zero_hbm, out_hbm, dst_v, e0_v, acc):
        c = lax.axis_index("c")
        s = lax.axis_index("s")
        w = c * NS + s
        pltpu.sync_copy(dst_hbm.at[w], dst_v)
        pltpu.sync_copy(e0_hbm, e0_v)
        pltpu.sync_copy(zero_hbm, acc.at[pl.ds(s * rp, rp)])
        plsc.subcore_barrier()

        @pl.loop(0, nchunk)
        def _(j):
            pltpu.sync_copy(e0_v, acc.at[dst_v.at[j]], add=True)

        plsc.subcore_barrier()
        pltpu.sync_copy(acc.at[pl.ds(s * rp, rp)], out_hbm.at[c, pl.ds(s * rp, rp)])

    return deg_kernel


def _make_prop(np_, rp, nchunk, dh):
    mesh = plsc.VectorSubcoreMesh(core_axis_name="c", subcore_axis_name="s")

    @functools.partial(
        pl.kernel,
        out_type=[jax.ShapeDtypeStruct((np_, dh), jnp.float32),
                  jax.ShapeDtypeStruct((np_, dh), jnp.float32)],
        mesh=mesh,
        compiler_params=pltpu.CompilerParams(use_tc_tiling_on_sc=False),
        scratch_types=[
            pltpu.VMEM((nchunk, CHUNK), jnp.int32),
            pltpu.VMEM((nchunk, CHUNK), jnp.int32),
            pltpu.VMEM((CHUNK, dh), jnp.float32),
            pltpu.VMEM((CHUNK, dh), jnp.float32),
            pltpu.SemaphoreType.DMA,
            pltpu.SemaphoreType.DMA,
            pltpu.VMEM_SHARED((np_, dh), jnp.float32),
        ],
    )
    def prop_kernel(ga_hbm, gb_hbm, src_hbm, dst_hbm, zero_hbm, sa_hbm, sb_hbm,
                    src_v, dst_v, buf0, buf1, sem0, sem1, acc):
        c = lax.axis_index("c")
        s = lax.axis_index("s")
        pltpu.sync_copy(src_hbm.at[s], src_v)
        pltpu.sync_copy(dst_hbm.at[s], dst_v)
        pltpu.sync_copy(zero_hbm, acc.at[pl.ds(s * rp, rp)])
        plsc.subcore_barrier()

        bufs = (buf0, buf1)
        sems = (sem0, sem1)

        def run(g_hbm, out_hbm):
            def g_start(j, b):
                pltpu.make_async_copy(g_hbm.at[src_v.at[j]], bufs[b], sems[b]).start()

            def g_wait(j, b):
                pltpu.make_async_copy(g_hbm.at[src_v.at[j]], bufs[b], sems[b]).wait()

            g_start(0, 0)

            @pl.loop(0, nchunk, step=2)
            def _(j):
                g_start(j + 1, 1)
                g_wait(j, 0)
                pltpu.sync_copy(bufs[0], acc.at[dst_v.at[j]], add=True)

                @pl.when(j + 2 < nchunk)
                def _():
                    g_start(j + 2, 0)

                g_wait(j + 1, 1)
                pltpu.sync_copy(bufs[1], acc.at[dst_v.at[j + 1]], add=True)

            plsc.subcore_barrier()
            pltpu.sync_copy(acc.at[pl.ds(s * rp, rp)],
                            out_hbm.at[pl.ds(s * rp, rp)])

        @pl.when(c == 0)
        def _():
            run(ga_hbm, sa_hbm)

        @pl.when(c == 1)
        def _():
            run(gb_hbm, sb_hbm)

    return prop_kernel


def _tc_first(np_, r, din, d, dh):
    def body(xr, w1r, dpr, gar, gbr, dsr):
        dp = dpr[...]
        deg = dp[0] + dp[1] + 1.0
        dsv = lax.rsqrt(deg)
        h = jnp.dot(xr[...], w1r[...], preferred_element_type=jnp.float32)
        g = h * dsv[:, None]
        gar[...] = g[:, :dh]
        gbr[...] = g[:, dh:]
        dsr[...] = dsv[:, None]

    return pl.pallas_call(
        body,
        grid=(np_ // r,),
        in_specs=[
            pl.BlockSpec((r, din), lambda i: (i, 0)),
            pl.BlockSpec((din, d), lambda i: (0, 0)),
            pl.BlockSpec((2, r), lambda i: (0, i)),
        ],
        out_specs=[
            pl.BlockSpec((r, dh), lambda i: (i, 0)),
            pl.BlockSpec((r, dh), lambda i: (i, 0)),
            pl.BlockSpec((r, 1), lambda i: (i, 0)),
        ],
        out_shape=[
            jax.ShapeDtypeStruct((np_, dh), jnp.float32),
            jax.ShapeDtypeStruct((np_, dh), jnp.float32),
            jax.ShapeDtypeStruct((np_, 1), jnp.float32),
        ],
    )


def _tc_mid(np_, r, d, dout, dh):
    def body(sar, sbr, gar, gbr, dsr, br, w2r, oar, obr):
        dsv = dsr[...]
        full = jnp.concatenate(
            [sar[...] + gar[...], sbr[...] + gbr[...]], axis=1)
        h = jnp.maximum(dsv * full + br[...], 0.0)
        g = jnp.dot(h, w2r[...], preferred_element_type=jnp.float32) * dsv
        oar[...] = g[:, :dh]
        obr[...] = g[:, dh:]

    return pl.pallas_call(
        body,
        grid=(np_ // r,),
        in_specs=[
            pl.BlockSpec((r, dh), lambda i: (i, 0)),
            pl.BlockSpec((r, dh), lambda i: (i, 0)),
            pl.BlockSpec((r, dh), lambda i: (i, 0)),
            pl.BlockSpec((r, dh), lambda i: (i, 0)),
            pl.BlockSpec((r, 1), lambda i: (i, 0)),
            pl.BlockSpec((1, d), lambda i: (0, 0)),
            pl.BlockSpec((d, dout), lambda i: (0, 0)),
        ],
        out_specs=[
            pl.BlockSpec((r, dh), lambda i: (i, 0)),
            pl.BlockSpec((r, dh), lambda i: (i, 0)),
        ],
        out_shape=[
            jax.ShapeDtypeStruct((np_, dh), jnp.float32),
            jax.ShapeDtypeStruct((np_, dh), jnp.float32),
        ],
    )


def _tc_last(np_, r, d, dh):
    def body(sar, sbr, gar, gbr, dsr, br, wlr, blr, outr):
        dsv = dsr[...]
        full = jnp.concatenate(
            [sar[...] + gar[...], sbr[...] + gbr[...]], axis=1)
        h = jnp.maximum(dsv * full + br[...], 0.0)
        outr[...] = jnp.dot(h, wlr[...], preferred_element_type=jnp.float32) + blr[...]

    return pl.pallas_call(
        body,
        grid=(np_ // r,),
        in_specs=[
            pl.BlockSpec((r, dh), lambda i: (i, 0)),
            pl.BlockSpec((r, dh), lambda i: (i, 0)),
            pl.BlockSpec((r, dh), lambda i: (i, 0)),
            pl.BlockSpec((r, dh), lambda i: (i, 0)),
            pl.BlockSpec((r, 1), lambda i: (i, 0)),
            pl.BlockSpec((1, d), lambda i: (0, 0)),
            pl.BlockSpec((d, 1), lambda i: (0, 0)),
            pl.BlockSpec((1, 1), lambda i: (0, 0)),
        ],
        out_specs=pl.BlockSpec((r, 1), lambda i: (i, 0)),
        out_shape=jax.ShapeDtypeStruct((np_, 1), jnp.float32),
    )


def kernel(x, edge_index, W1, b1, W2, b2, Wl, bl):
    n, din = x.shape
    e = edge_index.shape[1]
    d = W1.shape[1]
    dout = W2.shape[1]
    dh = d // 2

    rp = _round_up(-(-(n + 1) // NS), 128)
    np_ = rp * NS
    e_pad = _round_up(e, NW * CHUNK * 2)
    nchunk_d = e_pad // (NW * CHUNK)
    nchunk_p = e_pad // (NS * CHUNK)

    src = edge_index[0].astype(jnp.int32)
    dst = edge_index[1].astype(jnp.int32)
    fill = jnp.full((e_pad - e,), n, jnp.int32)
    srcp = jnp.concatenate([src, fill])
    dstp = jnp.concatenate([dst, fill])
    src_p = srcp.reshape(NS, nchunk_p, CHUNK)
    dst_p = dstp.reshape(NS, nchunk_p, CHUNK)
    dst_d = dstp.reshape(NW, nchunk_d, CHUNK)

    xp = jnp.pad(x, ((0, np_ - n), (0, 0)))
    e0 = jnp.zeros((CHUNK, 16), jnp.float32).at[:, 0].set(1.0)
    z16 = jnp.zeros((rp, 16), jnp.float32)
    zdh = jnp.zeros((rp, dh), jnp.float32)

    degp = _make_deg(np_, rp, nchunk_d)(dst_d, e0, z16)
    deg2 = degp[:, :, 0]

    g1a, g1b, ds = _tc_first(np_, rp, din, d, dh)(xp, W1, deg2)
    s1a, s1b = _make_prop(np_, rp, nchunk_p, dh)(g1a, g1b, src_p, dst_p, zdh)
    g2a, g2b = _tc_mid(np_, rp, d, dout, dh)(
        s1a, s1b, g1a, g1b, ds, b1.reshape(1, d), W2)
    s2a, s2b = _make_prop(np_, rp, nchunk_p, dout // 2)(
        g2a, g2b, src_p, dst_p, zdh)
    out = _tc_last(np_, rp, dout, dout // 2)(
        s2a, s2b, g2a, g2b, ds, b2.reshape(1, dout), Wl, bl.reshape(1, 1))
    return out[:n]


# speedup vs baseline: 12.3333x; 1.0354x over previous
import functools

import jax
import jax.numpy as jnp
from jax import lax
from jax.experimental import pallas as pl
from jax.experimental.pallas import tpu as pltpu
from jax.experimental.pallas import tpu_sc as plsc

NC = 2
NS = 16
NW = NC * NS
CHUNK = 128


def _round_up(a, m):
    return (a + m - 1) // m * m


def _make_deg(np_, rp, nchunk):
    mesh = plsc.VectorSubcoreMesh(core_axis_name="c", subcore_axis_name="s")

    @functools.partial(
        pl.kernel,
        out_type=jax.ShapeDtypeStruct((NC, np_, 16), jnp.float32),
        mesh=mesh,
        compiler_params=pltpu.CompilerParams(use_tc_tiling_on_sc=False),
        scratch_types=[
            pltpu.VMEM((nchunk, CHUNK), jnp.int32),
            pltpu.VMEM((CHUNK, 16), jnp.float32),
            pltpu.VMEM_SHARED((np_, 16), jnp.float32),
        ],
    )
    def deg_kernel(dst_hbm, e0_hbm, zero_hbm, out_hbm, dst_v, e0_v, acc):
        c = lax.axis_index("c")
        s = lax.axis_index("s")
        w = c * NS + s
        pltpu.sync_copy(dst_hbm.at[w], dst_v)
        pltpu.sync_copy(e0_hbm, e0_v)
        pltpu.sync_copy(zero_hbm, acc.at[pl.ds(s * rp, rp)])
        plsc.subcore_barrier()

        @pl.loop(0, nchunk)
        def _(j):
            pltpu.sync_copy(e0_v, acc.at[dst_v.at[j]], add=True)

        plsc.subcore_barrier()
        pltpu.sync_copy(acc.at[pl.ds(s * rp, rp)], out_hbm.at[c, pl.ds(s * rp, rp)])

    return deg_kernel


def _make_prop(np_, rp, nchunk, dh):
    mesh = plsc.VectorSubcoreMesh(core_axis_name="c", subcore_axis_name="s")

    @functools.partial(
        pl.kernel,
        out_type=[jax.ShapeDtypeStruct((np_, dh), jnp.float32),
                  jax.ShapeDtypeStruct((np_, dh), jnp.float32)],
        mesh=mesh,
        compiler_params=pltpu.CompilerParams(use_tc_tiling_on_sc=False),
        scratch_types=[
            pltpu.VMEM((nchunk, CHUNK), jnp.int32),
            pltpu.VMEM((nchunk, CHUNK), jnp.int32),
            [pltpu.VMEM((CHUNK, dh), jnp.float32) for _ in range(4)],
            [pltpu.SemaphoreType.DMA for _ in range(4)],
            pltpu.VMEM_SHARED((np_, dh), jnp.float32),
        ],
    )
    def prop_kernel(ga_hbm, gb_hbm, src_hbm, dst_hbm, zero_hbm, sa_hbm, sb_hbm,
                    src_v, dst_v, bufs, gsems, acc):
        c = lax.axis_index("c")
        s = lax.axis_index("s")
        pltpu.sync_copy(src_hbm.at[s], src_v)
        pltpu.sync_copy(dst_hbm.at[s], dst_v)
        pltpu.sync_copy(zero_hbm, acc.at[pl.ds(s * rp, rp)])
        plsc.subcore_barrier()

        def run(g_hbm, out_hbm):
            def g_start(j, b):
                pltpu.make_async_copy(
                    g_hbm.at[src_v.at[j]], bufs[b], gsems[b]).start()

            def g_wait(j, b):
                pltpu.make_async_copy(
                    g_hbm.at[src_v.at[j]], bufs[b], gsems[b]).wait()

            for b in range(3):
                g_start(b, b)

            @pl.loop(0, nchunk, step=4)
            def _(g):
                for b in range(4):
                    j = g + b
                    g_wait(j, b)
                    nb = (b + 3) % 4

                    @pl.when(j + 3 < nchunk)
                    def _():
                        g_start(j + 3, nb)

                    pltpu.sync_copy(bufs[b], acc.at[dst_v.at[j]], add=True)

            plsc.subcore_barrier()
            pltpu.sync_copy(acc.at[pl.ds(s * rp, rp)],
                            out_hbm.at[pl.ds(s * rp, rp)])

        @pl.when(c == 0)
        def _():
            run(ga_hbm, sa_hbm)

        @pl.when(c == 1)
        def _():
            run(gb_hbm, sb_hbm)

    return prop_kernel


def _tc_first(np_, r, din, d, dh):
    def body(xr, w1r, dpr, gar, gbr, dsr):
        dp = dpr[...]
        deg = dp[0] + dp[1] + 1.0
        dsv = lax.rsqrt(deg)
        h = jnp.dot(xr[...], w1r[...], preferred_element_type=jnp.float32)
        g = h * dsv[:, None]
        gar[...] = g[:, :dh]
        gbr[...] = g[:, dh:]
        dsr[...] = dsv[:, None]

    return pl.pallas_call(
        body,
        grid=(np_ // r,),
        in_specs=[
            pl.BlockSpec((r, din), lambda i: (i, 0)),
            pl.BlockSpec((din, d), lambda i: (0, 0)),
            pl.BlockSpec((2, r), lambda i: (0, i)),
        ],
        out_specs=[
            pl.BlockSpec((r, dh), lambda i: (i, 0)),
            pl.BlockSpec((r, dh), lambda i: (i, 0)),
            pl.BlockSpec((r, 1), lambda i: (i, 0)),
        ],
        out_shape=[
            jax.ShapeDtypeStruct((np_, dh), jnp.float32),
            jax.ShapeDtypeStruct((np_, dh), jnp.float32),
            jax.ShapeDtypeStruct((np_, 1), jnp.float32),
        ],
    )


def _tc_mid(np_, r, d, dout, dh):
    def body(sar, sbr, gar, gbr, dsr, br, w2r, oar, obr):
        dsv = dsr[...]
        full = jnp.concatenate(
            [sar[...] + gar[...], sbr[...] + gbr[...]], axis=1)
        h = jnp.maximum(dsv * full + br[...], 0.0)
        g = jnp.dot(h, w2r[...], preferred_element_type=jnp.float32) * dsv
        oar[...] = g[:, :dh]
        obr[...] = g[:, dh:]

    return pl.pallas_call(
        body,
        grid=(np_ // r,),
        in_specs=[
            pl.BlockSpec((r, dh), lambda i: (i, 0)),
            pl.BlockSpec((r, dh), lambda i: (i, 0)),
            pl.BlockSpec((r, dh), lambda i: (i, 0)),
            pl.BlockSpec((r, dh), lambda i: (i, 0)),
            pl.BlockSpec((r, 1), lambda i: (i, 0)),
            pl.BlockSpec((1, d), lambda i: (0, 0)),
            pl.BlockSpec((d, dout), lambda i: (0, 0)),
        ],
        out_specs=[
            pl.BlockSpec((r, dh), lambda i: (i, 0)),
            pl.BlockSpec((r, dh), lambda i: (i, 0)),
        ],
        out_shape=[
            jax.ShapeDtypeStruct((np_, dh), jnp.float32),
            jax.ShapeDtypeStruct((np_, dh), jnp.float32),
        ],
    )


def _tc_last(np_, r, d, dh):
    def body(sar, sbr, gar, gbr, dsr, br, wlr, blr, outr):
        dsv = dsr[...]
        full = jnp.concatenate(
            [sar[...] + gar[...], sbr[...] + gbr[...]], axis=1)
        h = jnp.maximum(dsv * full + br[...], 0.0)
        outr[...] = jnp.dot(h, wlr[...], preferred_element_type=jnp.float32) + blr[...]

    return pl.pallas_call(
        body,
        grid=(np_ // r,),
        in_specs=[
            pl.BlockSpec((r, dh), lambda i: (i, 0)),
            pl.BlockSpec((r, dh), lambda i: (i, 0)),
            pl.BlockSpec((r, dh), lambda i: (i, 0)),
            pl.BlockSpec((r, dh), lambda i: (i, 0)),
            pl.BlockSpec((r, 1), lambda i: (i, 0)),
            pl.BlockSpec((1, d), lambda i: (0, 0)),
            pl.BlockSpec((d, 1), lambda i: (0, 0)),
            pl.BlockSpec((1, 1), lambda i: (0, 0)),
        ],
        out_specs=pl.BlockSpec((r, 1), lambda i: (i, 0)),
        out_shape=jax.ShapeDtypeStruct((np_, 1), jnp.float32),
    )


def kernel(x, edge_index, W1, b1, W2, b2, Wl, bl):
    n, din = x.shape
    e = edge_index.shape[1]
    d = W1.shape[1]
    dout = W2.shape[1]
    dh = d // 2

    rp = _round_up(-(-(n + 1) // NS), 128)
    np_ = rp * NS
    e_pad = _round_up(e, NW * CHUNK * 2)
    nchunk_d = e_pad // (NW * CHUNK)
    nchunk_p = e_pad // (NS * CHUNK)

    src = edge_index[0].astype(jnp.int32)
    dst = edge_index[1].astype(jnp.int32)
    fill = jnp.full((e_pad - e,), n, jnp.int32)
    srcp = jnp.concatenate([src, fill])
    dstp = jnp.concatenate([dst, fill])
    src_p = srcp.reshape(NS, nchunk_p, CHUNK)
    dst_p = dstp.reshape(NS, nchunk_p, CHUNK)
    dst_d = dstp.reshape(NW, nchunk_d, CHUNK)

    xp = jnp.pad(x, ((0, np_ - n), (0, 0)))
    e0 = jnp.zeros((CHUNK, 16), jnp.float32).at[:, 0].set(1.0)
    z16 = jnp.zeros((rp, 16), jnp.float32)
    zdh = jnp.zeros((rp, dh), jnp.float32)

    degp = _make_deg(np_, rp, nchunk_d)(dst_d, e0, z16)
    deg2 = degp[:, :, 0]

    g1a, g1b, ds = _tc_first(np_, rp, din, d, dh)(xp, W1, deg2)
    s1a, s1b = _make_prop(np_, rp, nchunk_p, dh)(g1a, g1b, src_p, dst_p, zdh)
    g2a, g2b = _tc_mid(np_, rp, d, dout, dh)(
        s1a, s1b, g1a, g1b, ds, b1.reshape(1, d), W2)
    s2a, s2b = _make_prop(np_, rp, nchunk_p, dout // 2)(
        g2a, g2b, src_p, dst_p, zdh)
    out = _tc_last(np_, rp, dout, dout // 2)(
        s2a, s2b, g2a, g2b, ds, b2.reshape(1, dout), Wl, bl.reshape(1, 1))
    return out[:n]
